# Initial kernel scaffold; baseline (speedup 1.0000x reference)
import jax
import jax.numpy as jnp
from jax.experimental import pallas as pl


def kernel(feat, coord, lang_feat, lang_len, Wr1, br1, Wr2, br2, Wl1, bl1, Wl2, bl2, Wf1, bf1, Wf2, bf2):
    raise NotImplementedError("write your pallas kernel here")


# speedup vs baseline: 17.9485x; 17.9485x over previous
import functools

import jax
import jax.numpy as jnp
from jax import lax
from jax.experimental import pallas as pl
from jax.experimental.pallas import tpu as pltpu
from jax.experimental.pallas import tpu_sc as plsc

_F32 = jnp.float32
_GW = 128
_GWIN = 256


def _bdot(a, b):
    return jnp.dot(a.astype(jnp.bfloat16), b.astype(jnp.bfloat16),
                   preferred_element_type=_F32)


def _hdot(a, b, dims):
    return lax.dot_general(a, b, dimension_numbers=(dims, ((), ())),
                           precision=lax.Precision.HIGHEST,
                           preferred_element_type=_F32)


def _stage1_body(n, k1,
                 feat_ref, coord_ref, coordT_ref, lang_ref, mask_ref,
                 wf1_ref, bf1_ref, wf2_ref, bf2_ref,
                 wl1_ref, bl1_ref, wl2_ref, bl2_ref, wv_ref,
                 h_ref, vft_ref, idx_ref, dn_ref):
    s = pl.program_id(0)
    feat = feat_ref[0]
    lang = lang_ref[0]
    mask = mask_ref[0]

    t1 = jnp.maximum(_hdot(feat, wf1_ref[...], ((1,), (0,))) + bf1_ref[...], 0.0)
    ft = _hdot(t1, wf2_ref[...], ((1,), (0,))) + bf2_ref[...]
    t2 = jnp.maximum(_hdot(lang, wl1_ref[...], ((1,), (0,))) + bl1_ref[...], 0.0)
    lf = _hdot(t2, wl2_ref[...], ((1,), (0,))) + bl2_ref[...]

    scores = _hdot(ft, lf, ((1,), (1,)))
    m = jnp.max(scores, axis=1, keepdims=True)
    e = jnp.exp(scores - m)
    p = e / jnp.sum(e, axis=1, keepdims=True)
    p = p * mask
    p = p / (jnp.sum(p, axis=1, keepdims=True) + 1e-7)
    g = _hdot(p, lf, ((1,), (0,)))
    h = ft * g

    c = coord_ref[0]
    v = (c[:, 0:1] * wv_ref[0:1, :] + c[:, 1:2] * wv_ref[1:2, :]
         + c[:, 2:3] * wv_ref[2:3, :])
    h_ref[0] = h
    vft_ref[0] = jnp.concatenate([v, ft], axis=1)

    cT = coordT_ref[0]
    dx = c[:, 0:1] - cT[0:1, :]
    dy = c[:, 1:2] - cT[1:2, :]
    dz = c[:, 2:3] - cT[2:3, :]
    d = (dx * dx + dy * dy) + dz * dz

    iota = lax.broadcasted_iota(jnp.int32, (n, n), 1)
    icols, dcols = [], []
    for _ in range(k1):
        mv = jnp.min(d, axis=1, keepdims=True)
        ismin = d == mv
        jt = jnp.min(jnp.where(ismin, iota, n), axis=1, keepdims=True)
        icols.append(jt)
        dcols.append(mv)
        d = jnp.where(iota == jt, 1e30, d)
    idx_ref[0] = jnp.concatenate(icols, axis=1) + s * n
    dn_ref[0] = jnp.sqrt(jnp.concatenate(dcols, axis=1))


def _stage3_body(n, k1,
                 hug_ref, dn_ref, vft_ref,
                 wu_ref, wd_ref, br1_ref, wr2_ref, br2_ref,
                 out_ref, score_ref):
    hug = hug_ref[0]
    vft = vft_ref[0]
    v = vft[:, :64]
    ft = vft[:, 64:]
    dn = dn_ref[0]
    wd = wd_ref[...]
    l1s = []
    for t in range(k1):
        row = hug[t]
        ug = (row[:, 64:65] * wu_ref[0:1, :] + row[:, 65:66] * wu_ref[1:2, :]
              + row[:, 66:67] * wu_ref[2:3, :])
        l1s.append(ug + v + dn[:, t:t + 1] * wd + br1_ref[...])
    act = jnp.maximum(jnp.concatenate(l1s, axis=0), 0.0)
    rel = _bdot(act, wr2_ref[...]) + br2_ref[...]
    rel3 = rel.reshape(k1, n, 64)
    hg3 = hug[:, :, :64]
    acc = jnp.sum(rel3 * hg3, axis=0) + ft
    out_ref[0] = acc
    ones = jnp.ones((1, 64), dtype=_F32)
    score_ref[0] = _hdot(ones, acc, ((1,), (1,)))


def _sc_gather(table, idxflat):
    b = idxflat.shape[1]
    mesh = plsc.VectorSubcoreMesh(core_axis_name="c", subcore_axis_name="s")

    @functools.partial(
        pl.kernel, mesh=mesh,
        out_type=jax.ShapeDtypeStruct((b, _GW), _F32))
    def k(tab_hbm, idx_hbm, out_hbm):
        def body(idx_vmem, out_vmem):
            pltpu.sync_copy(tab_hbm.at[idx_vmem.at[0]], out_vmem)

        pltpu.emit_pipeline(
            body,
            grid=(b // _GWIN,),
            in_specs=[pl.BlockSpec((1, _GWIN), lambda i: (0, i))],
            out_specs=[pl.BlockSpec((_GWIN, _GW), lambda i: (i, 0))],
            core_axis_name=("c", "s"),
            dimension_semantics=(pltpu.PARALLEL,),
        )(idx_hbm, out_hbm)

    return k(table, idxflat)


def kernel(feat, coord, lang_feat, lang_len, Wr1, br1, Wr2, br2,
           Wl1, bl1, Wl2, bl2, Wf1, bf1, Wf2, bf2):
    s, n, pc_id = feat.shape
    _, l, lang_id = lang_feat.shape
    d = Wf1.shape[1]
    k1 = min(16, n - 1) + 1

    wa, wb, wc, wd = Wr1[0:3], Wr1[3:6], Wr1[6:9], Wr1[9:10]
    wu = wa + wc
    wv = wb - wc
    mask = (jnp.arange(l)[None, :] < lang_len[:, None]).astype(_F32)
    mask = mask.reshape(s, 1, l)
    coord_t = jnp.swapaxes(coord, 1, 2)

    b2 = lambda x: x.reshape(1, -1)
    full = lambda shape: pl.BlockSpec(shape, lambda i: tuple(0 for _ in shape))
    per_s = lambda shape: pl.BlockSpec((1,) + shape,
                                       lambda i: (i,) + tuple(0 for _ in shape))

    h, vft, idx, dn = pl.pallas_call(
        functools.partial(_stage1_body, n, k1),
        grid=(s,),
        in_specs=[
            per_s((n, pc_id)), per_s((n, 3)), per_s((3, n)),
            per_s((l, lang_id)), per_s((1, l)),
            full((pc_id, d)), full((1, d)), full((d, d)), full((1, d)),
            full((lang_id, d)), full((1, d)), full((d, d)), full((1, d)),
            full((3, d)),
        ],
        out_specs=[per_s((n, d)), per_s((n, 2 * d)),
                   per_s((n, k1)), per_s((n, k1))],
        out_shape=[
            jax.ShapeDtypeStruct((s, n, d), _F32),
            jax.ShapeDtypeStruct((s, n, 2 * d), _F32),
            jax.ShapeDtypeStruct((s, n, k1), jnp.int32),
            jax.ShapeDtypeStruct((s, n, k1), _F32),
        ],
    )(feat, coord, coord_t, lang_feat, mask,
      Wf1, b2(bf1), Wf2, b2(bf2), Wl1, b2(bl1), Wl2, b2(bl2), wv)

    pad = jnp.zeros((s, n, _GW - d - 3), _F32)
    table = jnp.concatenate([h, coord, pad], axis=-1).reshape(s * n, _GW)
    idxflat = jnp.swapaxes(idx, 1, 2).reshape(1, -1)

    gathered = _sc_gather(table, idxflat)
    hug = gathered.reshape(s, k1, n, _GW)

    out, score = pl.pallas_call(
        functools.partial(_stage3_body, n, k1),
        grid=(s,),
        in_specs=[
            per_s((k1, n, _GW)), per_s((n, k1)), per_s((n, 2 * d)),
            full((3, d)), full((1, d)), full((1, d)),
            full((d, d)), full((1, d)),
        ],
        out_specs=[per_s((n, d)), per_s((1, n))],
        out_shape=[
            jax.ShapeDtypeStruct((s, n, d), _F32),
            jax.ShapeDtypeStruct((s, 1, n), _F32),
        ],
    )(hug, dn, vft, wu, b2(wd[0]), b2(br1), Wr2, b2(br2))

    return out, score.reshape(s, n)


# speedup vs baseline: 21.6917x; 1.2086x over previous
import functools

import jax
import jax.numpy as jnp
from jax import lax
from jax.experimental import pallas as pl
from jax.experimental.pallas import tpu as pltpu
from jax.experimental.pallas import tpu_sc as plsc

_F32 = jnp.float32
_GW = 128
_GWIN = 256
_CHUNKS = 4


def _bdot(a, b):
    return jnp.dot(a.astype(jnp.bfloat16), b.astype(jnp.bfloat16),
                   preferred_element_type=_F32)


def _hdot(a, b, dims):
    return lax.dot_general(a, b, dimension_numbers=(dims, ((), ())),
                           precision=lax.Precision.HIGHEST,
                           preferred_element_type=_F32)


def _stage1_body(n, k1,
                 feat_ref, coord_ref, coordT_ref, lang_ref, mask_ref,
                 wf1_ref, bf1_ref, wf2_ref, bf2_ref,
                 wl1_ref, bl1_ref, wl2_ref, bl2_ref, wv_ref,
                 h_ref, vft_ref, idx_ref):
    s = pl.program_id(0)
    feat = feat_ref[0]
    lang = lang_ref[0]
    mask = mask_ref[0]

    t1 = jnp.maximum(_hdot(feat, wf1_ref[...], ((1,), (0,))) + bf1_ref[...], 0.0)
    ft = _hdot(t1, wf2_ref[...], ((1,), (0,))) + bf2_ref[...]
    t2 = jnp.maximum(_hdot(lang, wl1_ref[...], ((1,), (0,))) + bl1_ref[...], 0.0)
    lf = _hdot(t2, wl2_ref[...], ((1,), (0,))) + bl2_ref[...]

    scores = _hdot(ft, lf, ((1,), (1,)))
    m = jnp.max(scores, axis=1, keepdims=True)
    e = jnp.exp(scores - m)
    p = e / jnp.sum(e, axis=1, keepdims=True)
    p = p * mask
    p = p / (jnp.sum(p, axis=1, keepdims=True) + 1e-7)
    g = _hdot(p, lf, ((1,), (0,)))
    h = ft * g

    c = coord_ref[0]
    v = (c[:, 0:1] * wv_ref[0:1, :] + c[:, 1:2] * wv_ref[1:2, :]
         + c[:, 2:3] * wv_ref[2:3, :])
    h_ref[0] = h
    vft_ref[0] = jnp.concatenate([v, ft], axis=1)

    cT = coordT_ref[0]
    dx = c[:, 0:1] - cT[0:1, :]
    dy = c[:, 1:2] - cT[1:2, :]
    dz = c[:, 2:3] - cT[2:3, :]
    d = (dx * dx + dy * dy) + dz * dz

    iota = lax.broadcasted_iota(jnp.int32, (n, n), 1).astype(_F32)
    big_i = jnp.float32(n)
    icols = []
    for _ in range(k1):
        mv = jnp.min(d, axis=1, keepdims=True)
        jt = jnp.min(jnp.where(d == mv, iota, big_i), axis=1, keepdims=True)
        icols.append(jt)
        d = jnp.where(iota == jt, 1e30, d)
    idx_f = jnp.concatenate(icols, axis=1)
    idx_ref[0] = idx_f.astype(jnp.int32) + s * n


def _stage3_body(n, k1,
                 hug_ref, coord_ref, vft_ref,
                 wupad_ref, wd_ref, br1_ref, wr2_ref, br2_ref,
                 out_ref, score_ref):
    rows = hug_ref[0]
    vft = vft_ref[0]
    v = vft[:, :64]
    ft = vft[:, 64:]
    ci = coord_ref[0]

    ug = _bdot(rows, wupad_ref[...])
    ci3 = jnp.broadcast_to(ci[None], (k1, n, 3)).reshape(k1 * n, 3)
    dd = rows[:, 64:67] - ci3
    dn = jnp.sqrt(jnp.sum(dd * dd, axis=1, keepdims=True))
    vrep = jnp.broadcast_to(v[None], (k1, n, 64)).reshape(k1 * n, 64)
    l1 = ug + vrep + dn * wd_ref[...] + br1_ref[...]
    act = jnp.maximum(l1, 0.0)
    rel = _bdot(act, wr2_ref[...]) + br2_ref[...]
    hg = rows[:, :64]
    acc = jnp.sum((rel * hg).reshape(k1, n, 64), axis=0) + ft
    out_ref[0] = acc
    ones = jnp.ones((1, 64), dtype=_F32)
    score_ref[0] = _hdot(ones, acc, ((1,), (1,)))


def _sc_gather(table, idxflat):
    b = idxflat.shape[1]
    mesh = plsc.VectorSubcoreMesh(core_axis_name="c", subcore_axis_name="s")

    @functools.partial(
        pl.kernel, mesh=mesh,
        out_type=jax.ShapeDtypeStruct((b, _GW), _F32))
    def k(tab_hbm, idx_hbm, out_hbm):
        def body(idx_vmem, out_vmem):
            pltpu.sync_copy(tab_hbm.at[idx_vmem.at[0]], out_vmem)

        pltpu.emit_pipeline(
            body,
            grid=(b // _GWIN,),
            in_specs=[pl.BlockSpec((1, _GWIN), lambda i: (0, i))],
            out_specs=[pl.BlockSpec((_GWIN, _GW), lambda i: (i, 0))],
            core_axis_name=("c", "s"),
            dimension_semantics=(pltpu.PARALLEL,),
        )(idx_hbm, out_hbm)

    return k(table, idxflat)


def _chunk(feat, coord, coord_t, lang_feat, mask,
           wupad, wv, wd, br1, Wr2, br2,
           Wl1, bl1, Wl2, bl2, Wf1, bf1, Wf2, bf2):
    s, n, pc_id = feat.shape
    _, l, lang_id = lang_feat.shape
    d = Wf1.shape[1]
    k1 = min(16, n - 1) + 1

    b2 = lambda x: x.reshape(1, -1)
    full = lambda shape: pl.BlockSpec(shape, lambda i: tuple(0 for _ in shape))
    per_s = lambda shape: pl.BlockSpec((1,) + shape,
                                       lambda i: (i,) + tuple(0 for _ in shape))

    h, vft, idx = pl.pallas_call(
        functools.partial(_stage1_body, n, k1),
        grid=(s,),
        in_specs=[
            per_s((n, pc_id)), per_s((n, 3)), per_s((3, n)),
            per_s((l, lang_id)), per_s((1, l)),
            full((pc_id, d)), full((1, d)), full((d, d)), full((1, d)),
            full((lang_id, d)), full((1, d)), full((d, d)), full((1, d)),
            full((3, d)),
        ],
        out_specs=[per_s((n, d)), per_s((n, 2 * d)), per_s((n, k1))],
        out_shape=[
            jax.ShapeDtypeStruct((s, n, d), _F32),
            jax.ShapeDtypeStruct((s, n, 2 * d), _F32),
            jax.ShapeDtypeStruct((s, n, k1), jnp.int32),
        ],
    )(feat, coord, coord_t, lang_feat, mask,
      Wf1, b2(bf1), Wf2, b2(bf2), Wl1, b2(bl1), Wl2, b2(bl2), wv)

    pad = jnp.zeros((s, n, _GW - d - 3), _F32)
    table = jnp.concatenate([h, coord, pad], axis=-1).reshape(s * n, _GW)
    idxflat = jnp.swapaxes(idx, 1, 2).reshape(1, -1)

    gathered = _sc_gather(table, idxflat)
    hug = gathered.reshape(s, k1 * n, _GW)

    out, score = pl.pallas_call(
        functools.partial(_stage3_body, n, k1),
        grid=(s,),
        in_specs=[
            per_s((n * k1, _GW)), per_s((n, 3)), per_s((n, 2 * d)),
            full((_GW, d)), full((1, d)), full((1, d)),
            full((d, d)), full((1, d)),
        ],
        out_specs=[per_s((n, d)), per_s((1, n))],
        out_shape=[
            jax.ShapeDtypeStruct((s, n, d), _F32),
            jax.ShapeDtypeStruct((s, 1, n), _F32),
        ],
    )(hug, coord, vft, wupad, b2(wd[0]), b2(br1), Wr2, b2(br2))

    return out, score.reshape(s, n)


def kernel(feat, coord, lang_feat, lang_len, Wr1, br1, Wr2, br2,
           Wl1, bl1, Wl2, bl2, Wf1, bf1, Wf2, bf2):
    s, n, _ = feat.shape
    _, l, _ = lang_feat.shape
    d = Wf1.shape[1]

    wa, wb, wc, wd = Wr1[0:3], Wr1[3:6], Wr1[6:9], Wr1[9:10]
    wupad = jnp.zeros((_GW, d), _F32).at[64:67, :].set(wa + wc)
    wv = wb - wc
    mask = (jnp.arange(l)[None, :] < lang_len[:, None]).astype(_F32)
    mask = mask.reshape(s, 1, l)
    coord_t = jnp.swapaxes(coord, 1, 2)

    cs = s // _CHUNKS
    outs, scores = [], []
    for c in range(_CHUNKS):
        sl = slice(c * cs, (c + 1) * cs)
        o, sc = _chunk(feat[sl], coord[sl], coord_t[sl], lang_feat[sl],
                       mask[sl], wupad, wv, wd, br1, Wr2, br2,
                       Wl1, bl1, Wl2, bl2, Wf1, bf1, Wf2, bf2)
        outs.append(o)
        scores.append(sc)
    return jnp.concatenate(outs, 0), jnp.concatenate(scores, 0)
